# baseline (speedup 1.0000x reference)
import functools

import jax
import jax.numpy as jnp
from jax import lax
from jax.experimental import pallas as pl
from jax.experimental.pallas import tpu as pltpu
from jax.experimental.pallas import tpu_sc as plsc

EMB_DIM = 64
HID_DIM = 128
GATHER_WINDOW = 128
MLP_BLOCK = 2048


def _sc_user_gather(user_pairs, uidx):
    batch = uidx.shape[-1]
    grid = batch // GATHER_WINDOW
    mesh = plsc.VectorSubcoreMesh(core_axis_name="c", subcore_axis_name="s")

    @functools.partial(
        pl.kernel,
        out_type=jax.ShapeDtypeStruct((batch, 2 * EMB_DIM), jnp.float32),
        mesh=mesh,
    )
    def gather_kernel(ut_hbm, ui_hbm, uo_hbm):
        def body(ui_v, uo_v):
            pltpu.sync_copy(ut_hbm.at[ui_v.at[0]], uo_v)

        pltpu.emit_pipeline(
            body,
            grid=(grid,),
            in_specs=[pl.BlockSpec((1, GATHER_WINDOW), lambda i: (0, i))],
            out_specs=[
                pl.BlockSpec((GATHER_WINDOW, 2 * EMB_DIM), lambda i: (i, 0)),
            ],
            core_axis_name=("c", "s"),
            dimension_semantics=(pltpu.PARALLEL,),
        )(ui_hbm, uo_hbm)

    return gather_kernel(user_pairs, uidx)


def _item_gather_body(ids_ref, table_ref, out_ref):
    def step(i, _):
        idx = ids_ref[i]
        out_ref[pl.ds(i, 1), :] = table_ref[pl.ds(idx, 1), :]
        return 0

    lax.fori_loop(0, out_ref.shape[0], step, 0)


def _tc_item_gather(table, ids):
    batch = ids.shape[0]
    n = table.shape[0]
    return pl.pallas_call(
        _item_gather_body,
        in_specs=[
            pl.BlockSpec(memory_space=pltpu.SMEM),
            pl.BlockSpec((n, EMB_DIM), lambda: (0, 0)),
        ],
        out_specs=pl.BlockSpec((batch, EMB_DIM), lambda: (0, 0)),
        out_shape=jax.ShapeDtypeStruct((batch, EMB_DIM), jnp.float32),
    )(ids, table)


def _mlp_body(up_ref, ie_ref, upar_ref, w1u_ref, w1i_ref, b1_ref, w2_ref,
              b2_ref, w3_ref, b3_ref, out_ref):
    up = up_ref[...]
    ue = jnp.where(upar_ref[...][:, None] > 0, up[:, EMB_DIM:],
                   up[:, :EMB_DIM]).astype(jnp.float32)
    h = jnp.dot(ue, w1u_ref[...], preferred_element_type=jnp.float32)
    h += jnp.dot(ie_ref[...], w1i_ref[...], preferred_element_type=jnp.float32)
    h = jnp.maximum(h + b1_ref[...], 0.0)
    h = jnp.dot(h, w2_ref[...], preferred_element_type=jnp.float32)
    h = jnp.maximum(h + b2_ref[...], 0.0)
    logit = jnp.sum(h * w3_ref[...], axis=-1) + b3_ref[0]
    p = jax.nn.sigmoid(logit)
    out_ref[...] = jnp.clip(p, 0.01, 0.99)


def _tc_mlp(user_pairs, item_emb, upar, W1, b1, W2, b2, W3, b3):
    batch = user_pairs.shape[0]
    w1u = W1[:EMB_DIM]
    w1i = W1[EMB_DIM:]
    w3r = jnp.reshape(W3, (1, HID_DIM // 2))
    b1r = jnp.reshape(b1, (1, HID_DIM))
    b2r = jnp.reshape(b2, (1, HID_DIM // 2))
    grid = batch // MLP_BLOCK
    rep = lambda i: (0, 0)
    return pl.pallas_call(
        _mlp_body,
        grid=(grid,),
        in_specs=[
            pl.BlockSpec((MLP_BLOCK, 2 * EMB_DIM), lambda i: (i, 0)),
            pl.BlockSpec((MLP_BLOCK, EMB_DIM), lambda i: (i, 0)),
            pl.BlockSpec((MLP_BLOCK,), lambda i: (i,)),
            pl.BlockSpec((EMB_DIM, HID_DIM), rep),
            pl.BlockSpec((EMB_DIM, HID_DIM), rep),
            pl.BlockSpec((1, HID_DIM), rep),
            pl.BlockSpec((HID_DIM, HID_DIM // 2), rep),
            pl.BlockSpec((1, HID_DIM // 2), rep),
            pl.BlockSpec((1, HID_DIM // 2), rep),
            pl.BlockSpec((1,), lambda i: (0,)),
        ],
        out_specs=pl.BlockSpec((MLP_BLOCK,), lambda i: (i,)),
        out_shape=jax.ShapeDtypeStruct((batch,), jnp.float32),
    )(user_pairs, item_emb, upar, w1u, w1i, b1r, W2, b2r, w3r, b3)


def kernel(user_ids, item_ids, user_table, item_table, W1, b1, W2, b2, W3, b3):
    batch = user_ids.shape[0]
    uids = user_ids.astype(jnp.int32)
    iids = item_ids.astype(jnp.int32)
    upairs_idx = jnp.reshape(uids >> 1, (1, batch))
    upar = (uids & 1).astype(jnp.float32)
    ut2 = jnp.reshape(user_table, (user_table.shape[0] // 2, 2 * EMB_DIM))
    item_emb = _tc_item_gather(item_table, iids)
    user_pairs = _sc_user_gather(ut2, upairs_idx)
    return _tc_mlp(user_pairs, item_emb, upar, W1, b1, W2, b2, W3, b3)


# speedup vs baseline: 1.1487x; 1.1487x over previous
import functools

import jax
import jax.numpy as jnp
from jax.experimental import pallas as pl
from jax.experimental.pallas import tpu as pltpu
from jax.experimental.pallas import tpu_sc as plsc

EMB_DIM = 64
HID_DIM = 128
GATHER_WINDOW = 128
MLP_BLOCK = 2048


def _sc_double_gather(user_pairs, item_pairs, uids, iids):
    batch = uids.shape[-1]
    grid = batch // GATHER_WINDOW
    mesh = plsc.VectorSubcoreMesh(core_axis_name="c", subcore_axis_name="s")

    @functools.partial(
        pl.kernel,
        out_type=(
            jax.ShapeDtypeStruct((batch, 2 * EMB_DIM), jnp.float32),
            jax.ShapeDtypeStruct((batch, 2 * EMB_DIM), jnp.float32),
        ),
        mesh=mesh,
    )
    def gather_kernel(ut_hbm, it_hbm, ui_hbm, ii_hbm, uo_hbm, io_hbm):
        def body(ui_v, ii_v, uo_v, io_v):
            pltpu.sync_copy(ut_hbm.at[ui_v.at[0]], uo_v)
            pltpu.sync_copy(it_hbm.at[ii_v.at[0]], io_v)

        pltpu.emit_pipeline(
            body,
            grid=(grid,),
            in_specs=[
                pl.BlockSpec((1, GATHER_WINDOW), lambda i: (0, i)),
                pl.BlockSpec((1, GATHER_WINDOW), lambda i: (0, i)),
            ],
            out_specs=[
                pl.BlockSpec((GATHER_WINDOW, 2 * EMB_DIM), lambda i: (i, 0)),
                pl.BlockSpec((GATHER_WINDOW, 2 * EMB_DIM), lambda i: (i, 0)),
            ],
            core_axis_name=("c", "s"),
            dimension_semantics=(pltpu.PARALLEL,),
        )(ui_hbm, ii_hbm, uo_hbm, io_hbm)

    return gather_kernel(user_pairs, item_pairs, uids, iids)


def _mlp_body(up_ref, ip_ref, upar_ref, ipar_ref, w1u_ref, w1i_ref, b1_ref,
              w2_ref, b2_ref, w3_ref, b3_ref, out_ref):
    up = up_ref[...]
    ip = ip_ref[...]
    ue = jnp.where(upar_ref[...][:, None] > 0, up[:, EMB_DIM:], up[:, :EMB_DIM])
    ie = jnp.where(ipar_ref[...][:, None] > 0, ip[:, EMB_DIM:], ip[:, :EMB_DIM])
    h = jnp.dot(ue, w1u_ref[...], preferred_element_type=jnp.float32)
    h += jnp.dot(ie, w1i_ref[...], preferred_element_type=jnp.float32)
    h = jnp.maximum(h + b1_ref[...], 0.0)
    h = jnp.dot(h, w2_ref[...], preferred_element_type=jnp.float32)
    h = jnp.maximum(h + b2_ref[...], 0.0)
    logit = jnp.sum(h * w3_ref[...], axis=-1) + b3_ref[0]
    p = jax.nn.sigmoid(logit)
    out_ref[...] = jnp.clip(p, 0.01, 0.99)


def _tc_mlp(user_pairs, item_pairs, upar, ipar, W1, b1, W2, b2, W3, b3):
    batch = user_pairs.shape[0]
    w1u = W1[:EMB_DIM]
    w1i = W1[EMB_DIM:]
    w3r = jnp.reshape(W3, (1, HID_DIM // 2))
    b1r = jnp.reshape(b1, (1, HID_DIM))
    b2r = jnp.reshape(b2, (1, HID_DIM // 2))
    grid = batch // MLP_BLOCK
    rep = lambda i: (0, 0)
    return pl.pallas_call(
        _mlp_body,
        grid=(grid,),
        in_specs=[
            pl.BlockSpec((MLP_BLOCK, 2 * EMB_DIM), lambda i: (i, 0)),
            pl.BlockSpec((MLP_BLOCK, 2 * EMB_DIM), lambda i: (i, 0)),
            pl.BlockSpec((MLP_BLOCK,), lambda i: (i,)),
            pl.BlockSpec((MLP_BLOCK,), lambda i: (i,)),
            pl.BlockSpec((EMB_DIM, HID_DIM), rep),
            pl.BlockSpec((EMB_DIM, HID_DIM), rep),
            pl.BlockSpec((1, HID_DIM), rep),
            pl.BlockSpec((HID_DIM, HID_DIM // 2), rep),
            pl.BlockSpec((1, HID_DIM // 2), rep),
            pl.BlockSpec((1, HID_DIM // 2), rep),
            pl.BlockSpec((1,), lambda i: (0,)),
        ],
        out_specs=pl.BlockSpec((MLP_BLOCK,), lambda i: (i,)),
        out_shape=jax.ShapeDtypeStruct((batch,), jnp.float32),
    )(user_pairs, item_pairs, upar, ipar, w1u, w1i, b1r, W2, b2r, w3r, b3)


def kernel(user_ids, item_ids, user_table, item_table, W1, b1, W2, b2, W3, b3):
    batch = user_ids.shape[0]
    uids32 = user_ids.astype(jnp.int32)
    iids32 = item_ids.astype(jnp.int32)
    upairs_idx = jnp.reshape(uids32 >> 1, (1, batch))
    ipairs_idx = jnp.reshape(iids32 >> 1, (1, batch))
    upar = (uids32 & 1).astype(jnp.float32)
    ipar = (iids32 & 1).astype(jnp.float32)
    ut2 = jnp.reshape(user_table, (user_table.shape[0] // 2, 2 * EMB_DIM))
    it2 = jnp.reshape(item_table, (item_table.shape[0] // 2, 2 * EMB_DIM))
    user_pairs, item_pairs = _sc_double_gather(ut2, it2, upairs_idx, ipairs_idx)
    return _tc_mlp(user_pairs, item_pairs, upar, ipar, W1, b1, W2, b2, W3, b3)
